# baseline (speedup 1.0000x reference)
import functools

import jax
import jax.numpy as jnp
from jax import lax
from jax.experimental import pallas as pl
from jax.experimental.pallas import tpu as pltpu
from jax.experimental.pallas import tpu_sc as plsc

BATCH = 4096
SEQ = 200
EMB = 64
HID = 128
OUT = 3
LANES = 16


def _sc_gather_sum(tokens_flat, emb_table):
    info = plsc.get_sparse_core_info()
    nc, ns = info.num_cores, info.num_subcores
    nw = nc * ns
    rows_per_w = BATCH // nw
    tok_per_w = rows_per_w * SEQ

    mesh = plsc.VectorSubcoreMesh(core_axis_name="c", subcore_axis_name="s")

    @functools.partial(
        pl.kernel,
        out_type=jax.ShapeDtypeStruct((BATCH, EMB), jnp.float32),
        mesh=mesh,
        compiler_params=pltpu.CompilerParams(use_tc_tiling_on_sc=True),
        scratch_types=[
            pltpu.VMEM((tok_per_w + LANES,), jnp.int32),
            pltpu.VMEM((SEQ, EMB), jnp.float32),
            pltpu.VMEM((SEQ, EMB), jnp.float32),
            pltpu.VMEM((rows_per_w, EMB), jnp.float32),
            pltpu.SemaphoreType.DMA,
            pltpu.SemaphoreType.DMA,
            pltpu.SemaphoreType.DMA,
        ],
    )
    def k(tok_hbm, table_hbm, out_hbm,
          tok_v, buf0, buf1, out_v,
          tok_sem, sem0, sem1):
        wid = lax.axis_index("s") * nc + lax.axis_index("c")
        base = wid * tok_per_w
        pltpu.async_copy(
            tok_hbm.at[pl.ds(base, tok_per_w)],
            tok_v.at[pl.ds(0, tok_per_w)], tok_sem).wait()

        bufs = ((buf0, sem0), (buf1, sem1))

        n_full = SEQ // LANES
        n_tail = SEQ % LANES

        def issue_chunk(buf, sem, off, jbase, count):
            vec = tok_v[pl.ds(off + jbase, LANES)]
            for l in range(count):
                pltpu.async_copy(
                    table_hbm.at[pl.ds(vec[l], 1)],
                    buf.at[pl.ds(jbase + l, 1)], sem)

        def issue_row(r, slot):
            buf, sem = bufs[slot]
            off = r * SEQ

            def body(g, _):
                issue_chunk(buf, sem, off, g * LANES, LANES)
                return 0
            lax.fori_loop(0, n_full, body, 0, unroll=4)
            issue_chunk(buf, sem, off, n_full * LANES, n_tail)

        def drain(slot):
            buf, sem = bufs[slot]
            pltpu.make_async_copy(table_hbm.at[pl.ds(0, SEQ)], buf, sem).wait()

        def accum16(buf, g, acc, count):
            a0, a1, a2, a3 = acc
            for jj in range(count):
                j = g * LANES + jj
                a0 = a0 + buf[j, pl.ds(0, LANES)]
                a1 = a1 + buf[j, pl.ds(LANES, LANES)]
                a2 = a2 + buf[j, pl.ds(2 * LANES, LANES)]
                a3 = a3 + buf[j, pl.ds(3 * LANES, LANES)]
            return (a0, a1, a2, a3)

        def accum_store(r, slot):
            buf, _ = bufs[slot]
            zero = jnp.zeros((LANES,), jnp.float32)

            def body(g, acc):
                return accum16(buf, g, acc, LANES)
            acc = lax.fori_loop(0, n_full, body, (zero, zero, zero, zero))
            acc = accum16(buf, n_full, acc, n_tail)
            out_v[r, pl.ds(0, LANES)] = acc[0]
            out_v[r, pl.ds(LANES, LANES)] = acc[1]
            out_v[r, pl.ds(2 * LANES, LANES)] = acc[2]
            out_v[r, pl.ds(3 * LANES, LANES)] = acc[3]

        issue_row(0, 0)

        def outer(rr, _):
            r0 = rr * 2

            @pl.when(r0 + 1 < rows_per_w)
            def _():
                issue_row(r0 + 1, 1)
            drain(0)
            accum_store(r0, 0)

            @pl.when(r0 + 1 < rows_per_w)
            def _():
                @pl.when(r0 + 2 < rows_per_w)
                def _():
                    issue_row(r0 + 2, 0)
                drain(1)
                accum_store(r0 + 1, 1)
            return 0

        lax.fori_loop(0, rows_per_w // 2, outer, 0)
        pltpu.async_copy(
            out_v, out_hbm.at[pl.ds(wid * rows_per_w, rows_per_w)],
            tok_sem).wait()

    return k(tokens_flat, emb_table)


def _tc_mlp(sums, tokens, emb0, W1, b1, W2p, b2p):
    blk = 512
    grid = (BATCH // blk,)

    def body(sum_ref, tok_ref, emb0_ref, w1_ref, b1_ref, w2_ref, b2_ref,
             out_ref):
        tok = tok_ref[...]
        cnt = jnp.sum((tok != 0).astype(jnp.float32), axis=1, keepdims=True)
        n_zero = float(SEQ) - cnt
        corrected = sum_ref[...] - n_zero * emb0_ref[0:1, :]
        pooled = corrected / jnp.maximum(cnt, 1.0)
        h = jnp.dot(pooled, w1_ref[...], precision="highest") + b1_ref[...]
        h = jnp.maximum(h, 0.0)
        logits = jnp.dot(h, w2_ref[...], precision="highest") + b2_ref[...]
        m = jnp.max(logits, axis=-1, keepdims=True)
        e = jnp.exp(logits - m)
        out_ref[...] = e / jnp.sum(e, axis=-1, keepdims=True)

    return pl.pallas_call(
        body,
        grid=grid,
        in_specs=[
            pl.BlockSpec((blk, EMB), lambda i: (i, 0)),
            pl.BlockSpec((blk, SEQ), lambda i: (i, 0)),
            pl.BlockSpec((8, EMB), lambda i: (0, 0)),
            pl.BlockSpec((EMB, HID), lambda i: (0, 0)),
            pl.BlockSpec((1, HID), lambda i: (0, 0)),
            pl.BlockSpec((HID, HID), lambda i: (0, 0)),
            pl.BlockSpec((1, HID), lambda i: (0, 0)),
        ],
        out_specs=pl.BlockSpec((blk, HID), lambda i: (i, 0)),
        out_shape=jax.ShapeDtypeStruct((BATCH, HID), jnp.float32),
    )(sums, tokens, emb0, W1, b1, W2p, b2p)


def kernel(tokens, emb_table, W1, b1, W2, b2):
    tokens = tokens.astype(jnp.int32)
    sums = _sc_gather_sum(tokens.reshape(-1), emb_table)
    W2p = jnp.pad(W2, ((0, 0), (0, HID - OUT)))
    b2p = jnp.concatenate(
        [b2, jnp.full((HID - OUT,), -1e30, jnp.float32)]).reshape(1, HID)
    out_full = _tc_mlp(sums, tokens, emb_table, W1, b1.reshape(1, HID),
                       W2p, b2p)
    return out_full[:, :OUT]


# speedup vs baseline: 1.0004x; 1.0004x over previous
import functools

import jax
import jax.numpy as jnp
from jax import lax
from jax.experimental import pallas as pl
from jax.experimental.pallas import tpu as pltpu
from jax.experimental.pallas import tpu_sc as plsc

BATCH = 4096
SEQ = 200
EMB = 64
HID = 128
OUT = 3
LANES = 16
VOCAB = 1000000


def _sc_gather_sum(tokens_flat, emb_table):
    info = plsc.get_sparse_core_info()
    nc, ns = info.num_cores, info.num_subcores
    nw = nc * ns
    rows_per_w = BATCH // nw
    tok_per_w = rows_per_w * SEQ

    mesh = plsc.VectorSubcoreMesh(core_axis_name="c", subcore_axis_name="s")

    @functools.partial(
        pl.kernel,
        out_type=jax.ShapeDtypeStruct((BATCH, EMB), jnp.float32),
        mesh=mesh,
        compiler_params=pltpu.CompilerParams(use_tc_tiling_on_sc=True),
        scratch_types=[
            pltpu.VMEM((tok_per_w + LANES,), jnp.int32),
            pltpu.VMEM((SEQ, EMB), jnp.float32),
            pltpu.VMEM((SEQ, EMB), jnp.float32),
            pltpu.VMEM((rows_per_w, EMB), jnp.float32),
            pltpu.SemaphoreType.DMA,
            pltpu.SemaphoreType.DMA,
            pltpu.SemaphoreType.DMA,
        ],
    )
    def k(tok_hbm, table_hbm, out_hbm,
          tok_v, buf0, buf1, out_v,
          tok_sem, sem0, sem1):
        wid = lax.axis_index("s") * nc + lax.axis_index("c")
        base = wid * tok_per_w
        pltpu.async_copy(
            tok_hbm.at[pl.ds(base, tok_per_w)],
            tok_v.at[pl.ds(0, tok_per_w)], tok_sem).wait()

        bufs = ((buf0, sem0), (buf1, sem1))

        n_full = SEQ // LANES
        n_tail = SEQ % LANES

        def issue_chunk(buf, sem, off, jbase, count):
            vec = tok_v[pl.ds(off + jbase, LANES)]
            for l in range(count):
                pltpu.async_copy(
                    table_hbm.at[pl.ds(vec[l], 1)],
                    buf.at[pl.ds(jbase + l, 1)], sem)

        def issue_row(r, slot):
            buf, sem = bufs[slot]
            off = r * SEQ

            def body(g, _):
                issue_chunk(buf, sem, off, g * LANES, LANES)
                return 0
            lax.fori_loop(0, n_full, body, 0, unroll=4)
            issue_chunk(buf, sem, off, n_full * LANES, n_tail)

        def drain(slot):
            buf, sem = bufs[slot]
            pltpu.make_async_copy(table_hbm.at[pl.ds(0, SEQ)], buf, sem).wait()

        def accum16(buf, g, acc, count):
            a0, a1, a2, a3 = acc
            for jj in range(count):
                j = g * LANES + jj
                a0 = a0 + buf[j, pl.ds(0, LANES)]
                a1 = a1 + buf[j, pl.ds(LANES, LANES)]
                a2 = a2 + buf[j, pl.ds(2 * LANES, LANES)]
                a3 = a3 + buf[j, pl.ds(3 * LANES, LANES)]
            return (a0, a1, a2, a3)

        def accum_store(r, slot):
            buf, _ = bufs[slot]
            zero = jnp.zeros((LANES,), jnp.float32)

            def body(g, acc):
                return accum16(buf, g, acc, LANES)
            acc = lax.fori_loop(0, n_full, body, (zero, zero, zero, zero))
            acc = accum16(buf, n_full, acc, n_tail)
            out_v[r, pl.ds(0, LANES)] = acc[0]
            out_v[r, pl.ds(LANES, LANES)] = acc[1]
            out_v[r, pl.ds(2 * LANES, LANES)] = acc[2]
            out_v[r, pl.ds(3 * LANES, LANES)] = acc[3]

        issue_row(0, 0)

        def outer(rr, _):
            r0 = rr * 2

            @pl.when(r0 + 1 < rows_per_w)
            def _():
                issue_row(r0 + 1, 1)
            drain(0)
            accum_store(r0, 0)

            @pl.when(r0 + 1 < rows_per_w)
            def _():
                @pl.when(r0 + 2 < rows_per_w)
                def _():
                    issue_row(r0 + 2, 0)
                drain(1)
                accum_store(r0 + 1, 1)
            return 0

        lax.fori_loop(0, rows_per_w // 2, outer, 0)
        pltpu.async_copy(
            out_v, out_hbm.at[pl.ds(wid * rows_per_w, rows_per_w)],
            tok_sem).wait()

    return k(tokens_flat, emb_table)


def _tc_mlp(sums, tokens, emb0, W1, b1, W2p, b2p):
    blk = 512
    grid = (BATCH // blk,)

    def body(sum_ref, tok_ref, emb0_ref, w1_ref, b1_ref, w2_ref, b2_ref,
             out_ref):
        tok = tok_ref[...]
        cnt = jnp.sum((tok != 0).astype(jnp.float32), axis=1, keepdims=True)
        n_zero = float(SEQ) - cnt
        corrected = sum_ref[...] - n_zero * emb0_ref[0:1, :]
        pooled = corrected / jnp.maximum(cnt, 1.0)
        h = jnp.dot(pooled, w1_ref[...], precision="highest") + b1_ref[...]
        h = jnp.maximum(h, 0.0)
        logits = jnp.dot(h, w2_ref[...], precision="highest") + b2_ref[...]
        m = jnp.max(logits, axis=-1, keepdims=True)
        e = jnp.exp(logits - m)
        out_ref[...] = e / jnp.sum(e, axis=-1, keepdims=True)

    return pl.pallas_call(
        body,
        grid=grid,
        in_specs=[
            pl.BlockSpec((blk, EMB), lambda i: (i, 0)),
            pl.BlockSpec((blk, SEQ), lambda i: (i, 0)),
            pl.BlockSpec((8, EMB), lambda i: (0, 0)),
            pl.BlockSpec((EMB, HID), lambda i: (0, 0)),
            pl.BlockSpec((1, HID), lambda i: (0, 0)),
            pl.BlockSpec((HID, HID), lambda i: (0, 0)),
            pl.BlockSpec((1, HID), lambda i: (0, 0)),
        ],
        out_specs=pl.BlockSpec((blk, HID), lambda i: (i, 0)),
        out_shape=jax.ShapeDtypeStruct((BATCH, HID), jnp.float32),
    )(sums, tokens, emb0, W1, b1, W2p, b2p)


def kernel(tokens, emb_table, W1, b1, W2, b2):
    tokens = tokens.astype(jnp.int32)
    sums = _sc_gather_sum(tokens.reshape(-1), emb_table)
    W2p = jnp.pad(W2, ((0, 0), (0, HID - OUT)))
    b2p = jnp.concatenate(
        [b2, jnp.full((HID - OUT,), -1e30, jnp.float32)]).reshape(1, HID)
    out_full = _tc_mlp(sums, tokens, emb_table, W1, b1.reshape(1, HID),
                       W2p, b2p)
    return out_full[:, :OUT]
